# baseline (speedup 1.0000x reference)
import functools

import jax
import jax.numpy as jnp
from jax import lax
from jax.experimental import pallas as pl
from jax.experimental.pallas import tpu as pltpu
from jax.experimental.pallas import tpu_sc as plsc

NC = 2
NS = 16
K = 128


def _sc_degree(n_pad, niter):
    rpt = n_pad // NS
    half = niter // 2

    @functools.partial(
        pl.kernel,
        out_type=jax.ShapeDtypeStruct((NC, n_pad), jnp.float32),
        mesh=plsc.VectorSubcoreMesh(core_axis_name="c", subcore_axis_name="s"),
        scratch_types=[
            pltpu.VMEM((niter, K), jnp.int32),
            pltpu.VMEM((K,), jnp.float32),
            pltpu.VMEM_SHARED((n_pad,), jnp.float32),
        ],
    )
    def deg_kernel(dst_hbm, zero_hbm, one_hbm, deg_hbm, idx_v, ones_v, deg_sh):
        c = lax.axis_index("c")
        s = lax.axis_index("s")
        pltpu.sync_copy(zero_hbm, deg_sh.at[pl.ds(s * rpt, rpt)])
        pltpu.sync_copy(one_hbm, ones_v)
        pltpu.sync_copy(dst_hbm.at[s], idx_v)
        plsc.subcore_barrier()

        def body(j, carry):
            pltpu.sync_copy(ones_v, deg_sh.at[idx_v.at[j]], add=True)
            return carry

        lax.fori_loop(c * half, c * half + half, body, 0)
        plsc.subcore_barrier()
        pltpu.sync_copy(
            deg_sh.at[pl.ds(s * rpt, rpt)], deg_hbm.at[c, pl.ds(s * rpt, rpt)]
        )

    return deg_kernel


def _sc_scatter(n_pad, dh, niter):
    rpt = n_pad // NS

    @functools.partial(
        pl.kernel,
        out_type=jax.ShapeDtypeStruct((NC, n_pad, dh), jnp.float32),
        mesh=plsc.VectorSubcoreMesh(core_axis_name="c", subcore_axis_name="s"),
        scratch_types=[
            pltpu.VMEM((niter, K), jnp.int32),
            pltpu.VMEM((niter, K), jnp.int32),
            pltpu.VMEM((2 * K, dh), jnp.float32),
            pltpu.VMEM_SHARED((n_pad, dh), jnp.float32),
            pltpu.SemaphoreType.DMA,
        ],
        compiler_params=pltpu.CompilerParams(use_tc_tiling_on_sc=False),
    )
    def scatter_kernel(
        y_hbm, src_hbm, dst_hbm, zrow_hbm, acc_hbm,
        sidx, didx, rr, acc_sh, sem0,
    ):
        r0 = rr.at[pl.ds(0, K)]
        r1 = rr.at[pl.ds(K, K)]
        c = lax.axis_index("c")
        s = lax.axis_index("s")
        yc = y_hbm.at[c]
        pltpu.sync_copy(zrow_hbm, acc_sh.at[pl.ds(s * rpt, rpt)])
        pltpu.sync_copy(src_hbm.at[s], sidx)
        pltpu.sync_copy(dst_hbm.at[s], didx)
        plsc.subcore_barrier()

        pltpu.async_copy(yc.at[sidx.at[0]], r0, sem0)
        pltpu.async_copy(yc.at[sidx.at[1]], r1, sem0)

        def body(i, carry):
            j0 = 2 * i
            j1 = j0 + 1
            pltpu.make_async_copy(yc.at[pl.ds(0, 2 * K)], rr, sem0).wait()
            pltpu.sync_copy(r0, acc_sh.at[didx.at[j0]], add=True)

            @pl.when(i < niter // 2 - 1)
            def _():
                pltpu.async_copy(yc.at[sidx.at[j0 + 2]], r0, sem0)

            pltpu.sync_copy(r1, acc_sh.at[didx.at[j1]], add=True)

            @pl.when(i < niter // 2 - 1)
            def _():
                pltpu.async_copy(yc.at[sidx.at[j1 + 2]], r1, sem0)

            return carry

        lax.fori_loop(0, niter // 2, body, 0)
        plsc.subcore_barrier()
        pltpu.sync_copy(
            acc_sh.at[pl.ds(s * rpt, rpt)], acc_hbm.at[c, pl.ds(s * rpt, rpt)]
        )

    return scatter_kernel


def _tc_first(n, n_pad, d, dh, blk):
    def body(deg_ref, x_ref, w_ref, dinv_ref, y_ref):
        i = pl.program_id(0)
        dsum = deg_ref[0] + deg_ref[1] + 1.0
        rows = lax.broadcasted_iota(jnp.int32, (blk, 1), 0) + i * blk
        dinv = jnp.where(rows < n, lax.rsqrt(dsum), 0.0)
        dinv_ref[...] = dinv
        y = (
            jnp.dot(x_ref[...], w_ref[...], preferred_element_type=jnp.float32)
            * dinv
        )
        y_ref[0] = y[:, :dh]
        y_ref[1] = y[:, dh:]

    return pl.pallas_call(
        body,
        grid=(n_pad // blk,),
        in_specs=[
            pl.BlockSpec((NC, blk, 1), lambda i: (0, i, 0)),
            pl.BlockSpec((blk, d), lambda i: (i, 0)),
            pl.BlockSpec((d, d), lambda i: (0, 0)),
        ],
        out_specs=[
            pl.BlockSpec((blk, 1), lambda i: (i, 0)),
            pl.BlockSpec((NC, blk, dh), lambda i: (0, i, 0)),
        ],
        out_shape=[
            jax.ShapeDtypeStruct((n_pad, 1), jnp.float32),
            jax.ShapeDtypeStruct((NC, n_pad, dh), jnp.float32),
        ],
    )


def _tc_mid(n_pad, d, dh, blk):
    def body(acc_ref, y1_ref, dinv_ref, w_ref, b_ref, y2_ref):
        dinv = dinv_ref[...]
        a = jnp.concatenate([acc_ref[0] + y1_ref[0], acc_ref[1] + y1_ref[1]], axis=1)
        h = a * dinv + b_ref[...]
        y2 = jnp.dot(h, w_ref[...], preferred_element_type=jnp.float32) * dinv
        y2_ref[0] = y2[:, :dh]
        y2_ref[1] = y2[:, dh:]

    return pl.pallas_call(
        body,
        grid=(n_pad // blk,),
        in_specs=[
            pl.BlockSpec((NC, blk, dh), lambda i: (0, i, 0)),
            pl.BlockSpec((NC, blk, dh), lambda i: (0, i, 0)),
            pl.BlockSpec((blk, 1), lambda i: (i, 0)),
            pl.BlockSpec((d, d), lambda i: (0, 0)),
            pl.BlockSpec((1, d), lambda i: (0, 0)),
        ],
        out_specs=pl.BlockSpec((NC, blk, dh), lambda i: (0, i, 0)),
        out_shape=jax.ShapeDtypeStruct((NC, n_pad, dh), jnp.float32),
    )


def _tc_last(n, d, dh, blk):
    def body(acc_ref, y2_ref, dinv_ref, b_ref, out_ref):
        a = jnp.concatenate([acc_ref[0] + y2_ref[0], acc_ref[1] + y2_ref[1]], axis=1)
        out_ref[...] = a * dinv_ref[...] + b_ref[...]

    return pl.pallas_call(
        body,
        grid=(n // blk,),
        in_specs=[
            pl.BlockSpec((NC, blk, dh), lambda i: (0, i, 0)),
            pl.BlockSpec((NC, blk, dh), lambda i: (0, i, 0)),
            pl.BlockSpec((blk, 1), lambda i: (i, 0)),
            pl.BlockSpec((1, d), lambda i: (0, 0)),
        ],
        out_specs=pl.BlockSpec((blk, d), lambda i: (i, 0)),
        out_shape=jax.ShapeDtypeStruct((n, d), jnp.float32),
    )


def kernel(x, edge_index, W, b):
    n, d = x.shape
    e = edge_index.shape[1]
    dh = d // NC

    n_pad = ((n + 2047) // 2048) * 2048
    ept = (e + NS * K - 1) // (NS * K)
    niter = ept + (ept % 2)
    e_pad = NS * niter * K
    blk = 1024

    src = edge_index[0].astype(jnp.int32)
    dst = edge_index[1].astype(jnp.int32)
    pad = jnp.full((e_pad - e,), n, jnp.int32)
    src3 = jnp.concatenate([src, pad]).reshape(NS, niter, K)
    dst3 = jnp.concatenate([dst, pad]).reshape(NS, niter, K)
    x_p = jnp.pad(x, ((0, n_pad - n), (0, 0)))
    b2 = b.reshape(1, d)

    zero_deg = jnp.zeros((n_pad // NS,), jnp.float32)
    ones_k = jnp.ones((K,), jnp.float32)
    zero_rows = jnp.zeros((n_pad // NS, dh), jnp.float32)

    deg2 = _sc_degree(n_pad, niter)(dst3, zero_deg, ones_k)
    scatter = _sc_scatter(n_pad, dh, niter)

    dinv, y1 = _tc_first(n, n_pad, d, dh, blk)(deg2.reshape(NC, n_pad, 1), x_p, W)
    acc1 = scatter(y1, src3, dst3, zero_rows)
    y2 = _tc_mid(n_pad, d, dh, blk)(acc1, y1, dinv, W, b2)
    acc2 = scatter(y2, src3, dst3, zero_rows)
    return _tc_last(n, d, dh, 1000)(acc2, y2, dinv, b2)


# speedup vs baseline: 1.1166x; 1.1166x over previous
import functools

import jax
import jax.numpy as jnp
from jax import lax
from jax.experimental import pallas as pl
from jax.experimental.pallas import tpu as pltpu
from jax.experimental.pallas import tpu_sc as plsc

NC = 2
NS = 16
K = 128


def _sc_degree(n_pad, niter):
    rpt = n_pad // NS
    half = niter // 2

    @functools.partial(
        pl.kernel,
        out_type=jax.ShapeDtypeStruct((NC, n_pad), jnp.float32),
        mesh=plsc.VectorSubcoreMesh(core_axis_name="c", subcore_axis_name="s"),
        scratch_types=[
            pltpu.VMEM((niter, K), jnp.int32),
            pltpu.VMEM((K,), jnp.float32),
            pltpu.VMEM_SHARED((n_pad,), jnp.float32),
        ],
    )
    def deg_kernel(dst_hbm, zero_hbm, one_hbm, deg_hbm, idx_v, ones_v, deg_sh):
        c = lax.axis_index("c")
        s = lax.axis_index("s")
        pltpu.sync_copy(zero_hbm, deg_sh.at[pl.ds(s * rpt, rpt)])
        pltpu.sync_copy(one_hbm, ones_v)
        pltpu.sync_copy(dst_hbm.at[s], idx_v)
        plsc.subcore_barrier()

        def body(j, carry):
            pltpu.sync_copy(ones_v, deg_sh.at[idx_v.at[j]], add=True)
            return carry

        lax.fori_loop(c * half, c * half + half, body, 0)
        plsc.subcore_barrier()
        pltpu.sync_copy(
            deg_sh.at[pl.ds(s * rpt, rpt)], deg_hbm.at[c, pl.ds(s * rpt, rpt)]
        )

    return deg_kernel


def _sc_scatter(n_pad, dh, niter):
    rpt = n_pad // NS

    @functools.partial(
        pl.kernel,
        out_type=jax.ShapeDtypeStruct((NC, n_pad, dh), jnp.float32),
        mesh=plsc.VectorSubcoreMesh(core_axis_name="c", subcore_axis_name="s"),
        scratch_types=[
            pltpu.VMEM((niter, K), jnp.int32),
            pltpu.VMEM((niter, K), jnp.int32),
            pltpu.VMEM((K, dh), jnp.float32),
            pltpu.VMEM((K, dh), jnp.float32),
            pltpu.VMEM_SHARED((n_pad, dh), jnp.float32),
            pltpu.SemaphoreType.DMA,
            pltpu.SemaphoreType.DMA,
        ],
        compiler_params=pltpu.CompilerParams(use_tc_tiling_on_sc=False),
    )
    def scatter_kernel(
        y_hbm, src_hbm, dst_hbm, zrow_hbm, acc_hbm,
        sidx, didx, r0, r1, acc_sh, sem0, sem1,
    ):
        c = lax.axis_index("c")
        s = lax.axis_index("s")
        yc = y_hbm.at[c]
        pltpu.sync_copy(zrow_hbm, acc_sh.at[pl.ds(s * rpt, rpt)])
        pltpu.sync_copy(src_hbm.at[s], sidx)
        pltpu.sync_copy(dst_hbm.at[s], didx)
        plsc.subcore_barrier()

        pltpu.async_copy(yc.at[sidx.at[0]], r0, sem0)

        def body(i, carry):
            j0 = 2 * i
            j1 = j0 + 1
            pltpu.async_copy(yc.at[sidx.at[j1]], r1, sem1)
            pltpu.make_async_copy(yc.at[pl.ds(0, K)], r0, sem0).wait()
            pltpu.sync_copy(r0, acc_sh.at[didx.at[j0]], add=True)

            @pl.when(i < niter // 2 - 1)
            def _():
                pltpu.async_copy(yc.at[sidx.at[j0 + 2]], r0, sem0)

            pltpu.make_async_copy(yc.at[pl.ds(0, K)], r1, sem1).wait()
            pltpu.sync_copy(r1, acc_sh.at[didx.at[j1]], add=True)
            return carry

        lax.fori_loop(0, niter // 2, body, 0)
        plsc.subcore_barrier()
        pltpu.sync_copy(
            acc_sh.at[pl.ds(s * rpt, rpt)], acc_hbm.at[c, pl.ds(s * rpt, rpt)]
        )

    return scatter_kernel


def _tc_first(n, n_pad, d, dh, blk):
    def body(deg_ref, x_ref, w_ref, dinv_ref, y_ref):
        i = pl.program_id(0)
        dsum = deg_ref[0] + deg_ref[1] + 1.0
        rows = lax.broadcasted_iota(jnp.int32, (blk, 1), 0) + i * blk
        dinv = jnp.where(rows < n, lax.rsqrt(dsum), 0.0)
        dinv_ref[...] = dinv
        y = (
            jnp.dot(x_ref[...], w_ref[...], preferred_element_type=jnp.float32)
            * dinv
        )
        y_ref[0] = y[:, :dh]
        y_ref[1] = y[:, dh:]

    return pl.pallas_call(
        body,
        grid=(n_pad // blk,),
        in_specs=[
            pl.BlockSpec((NC, blk, 1), lambda i: (0, i, 0)),
            pl.BlockSpec((blk, d), lambda i: (i, 0)),
            pl.BlockSpec((d, d), lambda i: (0, 0)),
        ],
        out_specs=[
            pl.BlockSpec((blk, 1), lambda i: (i, 0)),
            pl.BlockSpec((NC, blk, dh), lambda i: (0, i, 0)),
        ],
        out_shape=[
            jax.ShapeDtypeStruct((n_pad, 1), jnp.float32),
            jax.ShapeDtypeStruct((NC, n_pad, dh), jnp.float32),
        ],
    )


def _tc_mid(n_pad, d, dh, blk):
    def body(acc_ref, y1_ref, dinv_ref, w_ref, b_ref, y2_ref):
        dinv = dinv_ref[...]
        a = jnp.concatenate([acc_ref[0] + y1_ref[0], acc_ref[1] + y1_ref[1]], axis=1)
        h = a * dinv + b_ref[...]
        y2 = jnp.dot(h, w_ref[...], preferred_element_type=jnp.float32) * dinv
        y2_ref[0] = y2[:, :dh]
        y2_ref[1] = y2[:, dh:]

    return pl.pallas_call(
        body,
        grid=(n_pad // blk,),
        in_specs=[
            pl.BlockSpec((NC, blk, dh), lambda i: (0, i, 0)),
            pl.BlockSpec((NC, blk, dh), lambda i: (0, i, 0)),
            pl.BlockSpec((blk, 1), lambda i: (i, 0)),
            pl.BlockSpec((d, d), lambda i: (0, 0)),
            pl.BlockSpec((1, d), lambda i: (0, 0)),
        ],
        out_specs=pl.BlockSpec((NC, blk, dh), lambda i: (0, i, 0)),
        out_shape=jax.ShapeDtypeStruct((NC, n_pad, dh), jnp.float32),
    )


def _tc_last(n, d, dh, blk):
    def body(acc_ref, y2_ref, dinv_ref, b_ref, out_ref):
        a = jnp.concatenate([acc_ref[0] + y2_ref[0], acc_ref[1] + y2_ref[1]], axis=1)
        out_ref[...] = a * dinv_ref[...] + b_ref[...]

    return pl.pallas_call(
        body,
        grid=(n // blk,),
        in_specs=[
            pl.BlockSpec((NC, blk, dh), lambda i: (0, i, 0)),
            pl.BlockSpec((NC, blk, dh), lambda i: (0, i, 0)),
            pl.BlockSpec((blk, 1), lambda i: (i, 0)),
            pl.BlockSpec((1, d), lambda i: (0, 0)),
        ],
        out_specs=pl.BlockSpec((blk, d), lambda i: (i, 0)),
        out_shape=jax.ShapeDtypeStruct((n, d), jnp.float32),
    )


def kernel(x, edge_index, W, b):
    n, d = x.shape
    e = edge_index.shape[1]
    dh = d // NC

    n_pad = ((n + 2047) // 2048) * 2048
    ept = (e + NS * K - 1) // (NS * K)
    niter = ept + (ept % 2)
    e_pad = NS * niter * K
    blk = 1024

    src = edge_index[0].astype(jnp.int32)
    dst = edge_index[1].astype(jnp.int32)
    pad = jnp.full((e_pad - e,), n, jnp.int32)
    src3 = jnp.concatenate([src, pad]).reshape(NS, niter, K)
    dst3 = jnp.concatenate([dst, pad]).reshape(NS, niter, K)
    x_p = jnp.pad(x, ((0, n_pad - n), (0, 0)))
    b2 = b.reshape(1, d)

    zero_deg = jnp.zeros((n_pad // NS,), jnp.float32)
    ones_k = jnp.ones((K,), jnp.float32)
    zero_rows = jnp.zeros((n_pad // NS, dh), jnp.float32)

    deg2 = _sc_degree(n_pad, niter)(dst3, zero_deg, ones_k)
    scatter = _sc_scatter(n_pad, dh, niter)

    dinv, y1 = _tc_first(n, n_pad, d, dh, blk)(deg2.reshape(NC, n_pad, 1), x_p, W)
    acc1 = scatter(y1, src3, dst3, zero_rows)
    y2 = _tc_mid(n_pad, d, dh, blk)(acc1, y1, dinv, W, b2)
    acc2 = scatter(y2, src3, dst3, zero_rows)
    return _tc_last(n, d, dh, 1000)(acc2, y2, dinv, b2)
